# Initial kernel scaffold; baseline (speedup 1.0000x reference)
import jax
import jax.numpy as jnp
from jax.experimental import pallas as pl


def kernel(x, edge_index, batch, W1, b1, W2, b2, W3, b3, fW1, fb1, fW2, fb2, fW3, fb3):
    raise NotImplementedError("write your pallas kernel here")


# speedup vs baseline: 8.9952x; 8.9952x over previous
import functools

import jax
import jax.numpy as jnp
from jax import lax
from jax.experimental import pallas as pl
from jax.experimental.pallas import tpu as pltpu
from jax.experimental.pallas import tpu_sc as plsc

NC = 2
NS = 16
NW = NC * NS
CHUNK = 80


def _make_deg(n, e):
    epw = e // NW
    nchunks = epw // CHUNK
    stri = -(-(n // NS) // 8) * 8
    last = n - (NS - 1) * stri
    mesh = plsc.VectorSubcoreMesh(core_axis_name="c", subcore_axis_name="s",
                                  num_cores=NC, num_subcores=NS)
    out_sds = jax.ShapeDtypeStruct((n, 128), jnp.float32)

    @functools.partial(
        pl.kernel,
        out_type=(out_sds, out_sds),
        mesh=mesh,
        scratch_types=[
            pltpu.VMEM((CHUNK,), jnp.int32),
            pltpu.VMEM((CHUNK, 128), jnp.float32),
            pltpu.VMEM_SHARED((n, 128), jnp.float32),
        ],
    )
    def deg_kernel(dst_hbm, zeros_hbm, ones_hbm, d0_hbm, d1_hbm,
                   di_v, ones_v, acc_sh):
        c = lax.axis_index("c")
        s = lax.axis_index("s")
        wid = s * NC + c
        ebase = wid * epw
        base = s * stri
        @pl.when(s < NS - 1)
        def _():
            pltpu.sync_copy(zeros_hbm, acc_sh.at[pl.ds(base, stri)])

        @pl.when(s == NS - 1)
        def _():
            pltpu.sync_copy(zeros_hbm.at[pl.ds(0, last)],
                            acc_sh.at[pl.ds(base, last)])

        pltpu.sync_copy(ones_hbm, ones_v)
        plsc.subcore_barrier()

        def chunk(g, carry):
            b = ebase + g * CHUNK
            pltpu.sync_copy(dst_hbm.at[pl.ds(b, CHUNK)], di_v)
            pltpu.sync_copy(ones_v, acc_sh.at[di_v], add=True)
            return carry

        lax.fori_loop(0, nchunks, chunk, 0)
        plsc.subcore_barrier()
        for cc, o_hbm in ((0, d0_hbm), (1, d1_hbm)):
            for sz, cond in ((stri, (c == cc) & (s < NS - 1)),
                             (last, (c == cc) & (s == NS - 1))):
                @pl.when(cond)
                def _(sz=sz, o_hbm=o_hbm):
                    pltpu.sync_copy(acc_sh.at[pl.ds(base, sz)],
                                    o_hbm.at[pl.ds(base, sz)])

    return deg_kernel


def _make_spmm(n, e, split_edges_by_core):
    epw = e // (NW if split_edges_by_core else NS)
    nchunks = epw // CHUNK
    stri = -(-(n // NS) // 8) * 8
    last = n - (NS - 1) * stri
    mesh = plsc.VectorSubcoreMesh(core_axis_name="c", subcore_axis_name="s",
                                  num_cores=NC, num_subcores=NS)
    out_sds = jax.ShapeDtypeStruct((n, 128), jnp.float32)

    @functools.partial(
        pl.kernel,
        out_type=(out_sds, out_sds),
        mesh=mesh,
        scratch_types=[
            pltpu.VMEM((CHUNK,), jnp.int32),
            pltpu.VMEM((CHUNK,), jnp.int32),
            pltpu.VMEM((CHUNK, 128), jnp.float32),
            pltpu.VMEM_SHARED((n, 128), jnp.float32),
            pltpu.SemaphoreType.DMA,
        ],
    )
    def spmm_kernel(ha_hbm, hb_hbm, src_hbm, dst_hbm, zeros_hbm,
                    oa_hbm, ob_hbm, si_v, di_v, rows_v, acc_sh, sem):
        c = lax.axis_index("c")
        s = lax.axis_index("s")
        if split_edges_by_core:
            ebase = (s * NC + c) * epw
        else:
            ebase = s * epw
        base = s * stri

        @pl.when(s < NS - 1)
        def _():
            pltpu.sync_copy(zeros_hbm, acc_sh.at[pl.ds(base, stri)])

        @pl.when(s == NS - 1)
        def _():
            pltpu.sync_copy(zeros_hbm.at[pl.ds(0, last)],
                            acc_sh.at[pl.ds(base, last)])

        plsc.subcore_barrier()

        def edge_loop(h_hbm):
            def chunk(g, carry):
                b = ebase + g * CHUNK
                pltpu.sync_copy(src_hbm.at[pl.ds(b, CHUNK)], si_v)
                pltpu.sync_copy(dst_hbm.at[pl.ds(b, CHUNK)], di_v)
                pltpu.async_copy(h_hbm.at[si_v], rows_v, sem).wait()
                pltpu.sync_copy(rows_v, acc_sh.at[di_v], add=True)
                return carry

            lax.fori_loop(0, nchunks, chunk, 0)

        @pl.when(c == 0)
        def _():
            edge_loop(ha_hbm)

        @pl.when(c == 1)
        def _():
            edge_loop(hb_hbm)

        plsc.subcore_barrier()
        for cc, o_hbm in ((0, oa_hbm), (1, ob_hbm)):
            for sz, cond in ((stri, (c == cc) & (s < NS - 1)),
                             (last, (c == cc) & (s == NS - 1))):
                @pl.when(cond)
                def _(sz=sz, o_hbm=o_hbm):
                    pltpu.sync_copy(acc_sh.at[pl.ds(base, sz)],
                                    o_hbm.at[pl.ds(base, sz)])

    return spmm_kernel


_R = 2000


def _scale0_body(d0, d1, x, xs_o, dis_o):
    deg = d0[:, :1] + d1[:, :1] + 1.0
    dis = lax.rsqrt(deg)
    dis_o[...] = dis
    xs_o[...] = x[...] * dis


def _tc_scale0(n, d_in):
    grid = n // _R
    return pl.pallas_call(
        _scale0_body,
        grid=(grid,),
        in_specs=[
            pl.BlockSpec((_R, 128), lambda i: (i, 0)),
            pl.BlockSpec((_R, 128), lambda i: (i, 0)),
            pl.BlockSpec((_R, d_in), lambda i: (i, 0)),
        ],
        out_specs=[
            pl.BlockSpec((_R, d_in), lambda i: (i, 0)),
            pl.BlockSpec((_R, 1), lambda i: (i, 0)),
        ],
        out_shape=[
            jax.ShapeDtypeStruct((n, d_in), jnp.float32),
            jax.ShapeDtypeStruct((n, 1), jnp.float32),
        ],
    )


def _l1_body(sa, sb, x, dis, w1, b1, w2, hw2_o, ha_o, hb_o):
    d = dis[...]
    q = d * (sa[...] + sb[...]) + (d * d) * x[...]
    h1 = jnp.maximum(
        lax.dot_general(q, w1[...], (((1,), (0,)), ((), ())),
                        preferred_element_type=jnp.float32) + b1[...], 0.0)
    hw2 = lax.dot_general(h1, w2[...], (((1,), (0,)), ((), ())),
                          preferred_element_type=jnp.float32)
    hw2_o[...] = hw2
    sc = d * hw2
    ha_o[...] = sc[:, :128]
    hb_o[...] = sc[:, 128:]


def _tc_l1(n, d_in, h):
    grid = n // _R
    return pl.pallas_call(
        _l1_body,
        grid=(grid,),
        in_specs=[
            pl.BlockSpec((_R, d_in), lambda i: (i, 0)),
            pl.BlockSpec((_R, d_in), lambda i: (i, 0)),
            pl.BlockSpec((_R, d_in), lambda i: (i, 0)),
            pl.BlockSpec((_R, 1), lambda i: (i, 0)),
            pl.BlockSpec((d_in, h), lambda i: (0, 0)),
            pl.BlockSpec((1, h), lambda i: (0, 0)),
            pl.BlockSpec((h, h), lambda i: (0, 0)),
        ],
        out_specs=[
            pl.BlockSpec((_R, h), lambda i: (i, 0)),
            pl.BlockSpec((_R, 128), lambda i: (i, 0)),
            pl.BlockSpec((_R, 128), lambda i: (i, 0)),
        ],
        out_shape=[
            jax.ShapeDtypeStruct((n, h), jnp.float32),
            jax.ShapeDtypeStruct((n, 128), jnp.float32),
            jax.ShapeDtypeStruct((n, 128), jnp.float32),
        ],
    )


def _mid_body(sa, sb, hw, dis, b, wn, hwn_o, ha_o, hb_o):
    d = dis[...]
    s = jnp.concatenate([sa[...], sb[...]], axis=1)
    h = jnp.maximum(d * s + (d * d) * hw[...] + b[...], 0.0)
    hwn = lax.dot_general(h, wn[...], (((1,), (0,)), ((), ())),
                          preferred_element_type=jnp.float32)
    hwn_o[...] = hwn
    sc = d * hwn
    ha_o[...] = sc[:, :128]
    hb_o[...] = sc[:, 128:]


def _tc_mid(n, h):
    grid = n // _R
    return pl.pallas_call(
        _mid_body,
        grid=(grid,),
        in_specs=[
            pl.BlockSpec((_R, 128), lambda i: (i, 0)),
            pl.BlockSpec((_R, 128), lambda i: (i, 0)),
            pl.BlockSpec((_R, h), lambda i: (i, 0)),
            pl.BlockSpec((_R, 1), lambda i: (i, 0)),
            pl.BlockSpec((1, h), lambda i: (0, 0)),
            pl.BlockSpec((h, h), lambda i: (0, 0)),
        ],
        out_specs=[
            pl.BlockSpec((_R, h), lambda i: (i, 0)),
            pl.BlockSpec((_R, 128), lambda i: (i, 0)),
            pl.BlockSpec((_R, 128), lambda i: (i, 0)),
        ],
        out_shape=[
            jax.ShapeDtypeStruct((n, h), jnp.float32),
            jax.ShapeDtypeStruct((n, 128), jnp.float32),
            jax.ShapeDtypeStruct((n, 128), jnp.float32),
        ],
    )


def _final_body(sa, sb, hw, dis, b, batch, fw1, fb1, fw2, fb2, fw3, fb3,
                out_o, g_acc):
    d = dis[...]
    s = jnp.concatenate([sa[...], sb[...]], axis=1)
    h3 = jnp.maximum(d * s + (d * d) * hw[...] + b[...], 0.0)
    gidx = lax.broadcasted_iota(jnp.int32, (_R, 16), 1)
    onehot = (batch[...] == gidx).astype(jnp.float32)
    part = lax.dot_general(onehot, h3, (((0,), (0,)), ((), ())),
                           preferred_element_type=jnp.float32)
    i = pl.program_id(0)

    @pl.when(i == 0)
    def _():
        g_acc[...] = jnp.zeros_like(g_acc)

    g_acc[...] += part

    @pl.when(i == pl.num_programs(0) - 1)
    def _():
        g = g_acc[...]
        z = jnp.maximum(
            lax.dot_general(g, fw1[...], (((1,), (0,)), ((), ())),
                            preferred_element_type=jnp.float32) + fb1[...], 0.0)
        z = jnp.maximum(
            lax.dot_general(z, fw2[...], (((1,), (0,)), ((), ())),
                            preferred_element_type=jnp.float32) + fb2[...], 0.0)
        z = lax.dot_general(z, fw3[...], (((1,), (0,)), ((), ())),
                            preferred_element_type=jnp.float32) + fb3[...]
        m = jnp.max(z, axis=-1, keepdims=True)
        lse = jnp.log(jnp.sum(jnp.exp(z - m), axis=-1, keepdims=True)) + m
        out_o[...] = z - lse


def _tc_final(n, h, c, g):
    grid = n // _R
    return pl.pallas_call(
        _final_body,
        grid=(grid,),
        in_specs=[
            pl.BlockSpec((_R, 128), lambda i: (i, 0)),
            pl.BlockSpec((_R, 128), lambda i: (i, 0)),
            pl.BlockSpec((_R, h), lambda i: (i, 0)),
            pl.BlockSpec((_R, 1), lambda i: (i, 0)),
            pl.BlockSpec((1, h), lambda i: (0, 0)),
            pl.BlockSpec((_R, 1), lambda i: (i, 0)),
            pl.BlockSpec((h, h), lambda i: (0, 0)),
            pl.BlockSpec((1, h), lambda i: (0, 0)),
            pl.BlockSpec((h, h // 2), lambda i: (0, 0)),
            pl.BlockSpec((1, h // 2), lambda i: (0, 0)),
            pl.BlockSpec((h // 2, c), lambda i: (0, 0)),
            pl.BlockSpec((1, c), lambda i: (0, 0)),
        ],
        out_specs=pl.BlockSpec((g, c), lambda i: (0, 0)),
        out_shape=jax.ShapeDtypeStruct((g, c), jnp.float32),
        scratch_shapes=[pltpu.VMEM((g, h), jnp.float32)],
    )


def kernel(x, edge_index, batch, W1, b1, W2, b2, W3, b3,
           fW1, fb1, fW2, fb2, fW3, fb3):
    n, d_in = x.shape
    e = edge_index.shape[1]
    h = W1.shape[1]
    c = fW3.shape[1]
    g = 16
    stri = -(-(n // NS) // 8) * 8

    src = edge_index[0]
    dst = edge_index[1]
    ones128 = jnp.ones((CHUNK, 128), jnp.float32)
    zeros128 = jnp.zeros((stri, 128), jnp.float32)
    batch2 = batch.reshape(n, 1)

    d0, d1 = _make_deg(n, e)(dst, zeros128, ones128)
    xs, dis = _tc_scale0(n, d_in)(d0, d1, x)

    s0a, s0b = _make_spmm(n, e, True)(xs, xs, src, dst, zeros128)
    hw2, h2a, h2b = _tc_l1(n, d_in, h)(s0a, s0b, x, dis, W1,
                                       b1.reshape(1, h), W2)

    s2a, s2b = _make_spmm(n, e, False)(h2a, h2b, src, dst, zeros128)
    hw3, h3a, h3b = _tc_mid(n, h)(s2a, s2b, hw2, dis, b2.reshape(1, h), W3)

    s3a, s3b = _make_spmm(n, e, False)(h3a, h3b, src, dst, zeros128)
    out = _tc_final(n, h, c, g)(s3a, s3b, hw3, dis, b3.reshape(1, h), batch2,
                                fW1, fb1.reshape(1, h), fW2,
                                fb2.reshape(1, h // 2), fW3,
                                fb3.reshape(1, c))
    return out


# speedup vs baseline: 20.1146x; 2.2362x over previous
import functools

import jax
import jax.numpy as jnp
from jax import lax
from jax.experimental import pallas as pl
from jax.experimental.pallas import tpu as pltpu
from jax.experimental.pallas import tpu_sc as plsc

NC = 2
NS = 16
NW = NC * NS
CHUNK = 80
SB = 32


def _split_chunks(total, workers, align=8):
    big = (-(-total // workers) + align - 1) // align * align
    for nbig in range(workers - 1, -1, -1):
        rem = total - big * nbig
        if rem < 0 or workers == nbig:
            continue
        sml, r = divmod(rem, workers - nbig)
        if r == 0 and sml % align == 0 and 0 < sml <= big:
            return big, sml, nbig
    raise ValueError((total, workers, align))


def _make_deg(n, e):
    total = e // CHUNK
    big, sml, nbig = _split_chunks(total, NW)
    stri = -(-(n // NS) // 8) * 8
    last = n - (NS - 1) * stri
    mesh = plsc.VectorSubcoreMesh(core_axis_name="c", subcore_axis_name="s",
                                  num_cores=NC, num_subcores=NS)
    out_sds = jax.ShapeDtypeStruct((n, 128), jnp.float32)

    @functools.partial(
        pl.kernel,
        out_type=(out_sds, out_sds),
        mesh=mesh,
        scratch_types=[
            pltpu.VMEM((big, CHUNK), jnp.int32),
            pltpu.VMEM((CHUNK, 128), jnp.float32),
            pltpu.VMEM_SHARED((n, 128), jnp.float32),
            pltpu.SemaphoreType.DMA,
            pltpu.SemaphoreType.DMA,
        ],
    )
    def deg_kernel(dst2_hbm, zeros_hbm, ones_hbm, d0_hbm, d1_hbm,
                   di_v, ones_v, acc_sh, ss0, ss1):
        c = lax.axis_index("c")
        s = lax.axis_index("s")
        w = s * NC + c
        base = s * stri
        ss = (ss0, ss1)

        @pl.when(s < NS - 1)
        def _():
            pltpu.sync_copy(zeros_hbm, acc_sh.at[pl.ds(base, stri)])

        @pl.when(s == NS - 1)
        def _():
            pltpu.sync_copy(zeros_hbm.at[pl.ds(0, last)],
                            acc_sh.at[pl.ds(base, last)])

        pltpu.sync_copy(ones_hbm, ones_v)

        @pl.when(w < nbig)
        def _():
            pltpu.sync_copy(dst2_hbm.at[pl.ds(w * big, big)], di_v)

        @pl.when(w >= nbig)
        def _():
            pltpu.sync_copy(
                dst2_hbm.at[pl.ds(nbig * big + (w - nbig) * sml, sml)],
                di_v.at[pl.ds(0, sml)])

        plsc.subcore_barrier()
        nch = jnp.where(w < nbig, big, sml)

        def body(t, carry):
            for b in (0, 1):
                j = 2 * t + b

                @pl.when(j >= 2)
                def _(b=b):
                    pltpu.make_async_copy(ones_hbm, ones_v, ss[b]).wait()

                pltpu.async_copy(ones_v, acc_sh.at[di_v.at[j]], ss[b],
                                 add=True)
            return carry

        lax.fori_loop(0, nch // 2, body, 0)
        pltpu.make_async_copy(ones_hbm, ones_v, ss0).wait()
        pltpu.make_async_copy(ones_hbm, ones_v, ss1).wait()
        plsc.subcore_barrier()
        for cc, o_hbm in ((0, d0_hbm), (1, d1_hbm)):
            for sz, cond in ((stri, (c == cc) & (s < NS - 1)),
                             (last, (c == cc) & (s == NS - 1))):
                @pl.when(cond)
                def _(sz=sz, o_hbm=o_hbm):
                    pltpu.sync_copy(acc_sh.at[pl.ds(base, sz)],
                                    o_hbm.at[pl.ds(base, sz)])

    return deg_kernel


def _make_spmm(n, e, split_edges_by_core):
    total = e // CHUNK
    big, sml, nbig = _split_chunks(total, NW if split_edges_by_core else NS,
                                   align=SB)
    stri = -(-(n // NS) // 8) * 8
    last = n - (NS - 1) * stri
    mesh = plsc.VectorSubcoreMesh(core_axis_name="c", subcore_axis_name="s",
                                  num_cores=NC, num_subcores=NS)
    out_sds = jax.ShapeDtypeStruct((n, 128), jnp.float32)

    @functools.partial(
        pl.kernel,
        out_type=(out_sds, out_sds),
        mesh=mesh,
        scratch_types=[
            pltpu.VMEM((SB, CHUNK), jnp.int32),
            pltpu.VMEM((SB, CHUNK), jnp.int32),
            pltpu.VMEM((CHUNK, 128), jnp.float32),
            pltpu.VMEM((CHUNK, 128), jnp.float32),
            pltpu.VMEM_SHARED((n, 128), jnp.float32),
            pltpu.SemaphoreType.DMA,
            pltpu.SemaphoreType.DMA,
            pltpu.SemaphoreType.DMA,
            pltpu.SemaphoreType.DMA,
        ],
    )
    def spmm_kernel(ha_hbm, hb_hbm, src2_hbm, dst2_hbm, zeros_hbm,
                    oa_hbm, ob_hbm, si_v, di_v, r0, r1, acc_sh,
                    sg0, sg1, ss0, ss1):
        c = lax.axis_index("c")
        s = lax.axis_index("s")
        w = s * NC + c if split_edges_by_core else s
        base = s * stri
        rows = (r0, r1)
        sg = (sg0, sg1)
        ss = (ss0, ss1)

        @pl.when(s < NS - 1)
        def _():
            pltpu.sync_copy(zeros_hbm, acc_sh.at[pl.ds(base, stri)])

        @pl.when(s == NS - 1)
        def _():
            pltpu.sync_copy(zeros_hbm.at[pl.ds(0, last)],
                            acc_sh.at[pl.ds(base, last)])

        plsc.subcore_barrier()
        nch = jnp.where(w < nbig, big, sml)
        rowbase = jnp.where(w < nbig, w * big,
                            nbig * big + (w - nbig) * sml)

        def edge_loop(h_hbm):
            def gather(j, b):
                pltpu.async_copy(h_hbm.at[si_v.at[j]], rows[b], sg[b])

            def gather_wait(b):
                pltpu.make_async_copy(h_hbm.at[pl.ds(0, CHUNK)], rows[b],
                                      sg[b]).wait()

            def scatter(j, b):
                pltpu.async_copy(rows[b], acc_sh.at[di_v.at[j]], ss[b],
                                 add=True)

            def scatter_wait(b):
                pltpu.make_async_copy(h_hbm.at[pl.ds(0, CHUNK)], rows[b],
                                      ss[b]).wait()

            def super_body(u, carry):
                rb = rowbase + u * SB
                pltpu.sync_copy(src2_hbm.at[pl.ds(rb, SB)], si_v)
                pltpu.sync_copy(dst2_hbm.at[pl.ds(rb, SB)], di_v)
                gather(0, 0)

                def body(t, carry2):
                    for b in (0, 1):
                        j = 2 * t + b

                        @pl.when(j >= 1)
                        def _(b=b):
                            scatter_wait(b ^ 1)

                        @pl.when(j + 1 < SB)
                        def _(j=j, b=b):
                            gather(j + 1, b ^ 1)

                        gather_wait(b)
                        scatter(j, b)
                    return carry2

                lax.fori_loop(0, SB // 2, body, 0)
                scatter_wait(1)
                return carry

            lax.fori_loop(0, nch // SB, super_body, 0)

        @pl.when(c == 0)
        def _():
            edge_loop(ha_hbm)

        @pl.when(c == 1)
        def _():
            edge_loop(hb_hbm)

        plsc.subcore_barrier()
        for cc, o_hbm in ((0, oa_hbm), (1, ob_hbm)):
            for sz, cond in ((stri, (c == cc) & (s < NS - 1)),
                             (last, (c == cc) & (s == NS - 1))):
                @pl.when(cond)
                def _(sz=sz, o_hbm=o_hbm):
                    pltpu.sync_copy(acc_sh.at[pl.ds(base, sz)],
                                    o_hbm.at[pl.ds(base, sz)])

    return spmm_kernel


_R = 2000


def _scale0_body(d0, d1, x, xs_o, dis_o):
    deg = d0[:, :1] + d1[:, :1] + 1.0
    dis = lax.rsqrt(deg)
    dis_o[...] = dis
    xs_o[...] = x[...] * dis


def _tc_scale0(n, d_in):
    grid = n // _R
    return pl.pallas_call(
        _scale0_body,
        grid=(grid,),
        in_specs=[
            pl.BlockSpec((_R, 128), lambda i: (i, 0)),
            pl.BlockSpec((_R, 128), lambda i: (i, 0)),
            pl.BlockSpec((_R, d_in), lambda i: (i, 0)),
        ],
        out_specs=[
            pl.BlockSpec((_R, d_in), lambda i: (i, 0)),
            pl.BlockSpec((_R, 1), lambda i: (i, 0)),
        ],
        out_shape=[
            jax.ShapeDtypeStruct((n, d_in), jnp.float32),
            jax.ShapeDtypeStruct((n, 1), jnp.float32),
        ],
    )


def _l1_body(sa, sb, x, dis, w1, b1, w2, hw2_o, ha_o, hb_o):
    d = dis[...]
    q = d * (sa[...] + sb[...]) + (d * d) * x[...]
    h1 = jnp.maximum(
        lax.dot_general(q, w1[...], (((1,), (0,)), ((), ())),
                        preferred_element_type=jnp.float32) + b1[...], 0.0)
    hw2 = lax.dot_general(h1, w2[...], (((1,), (0,)), ((), ())),
                          preferred_element_type=jnp.float32)
    hw2_o[...] = hw2
    sc = d * hw2
    ha_o[...] = sc[:, :128]
    hb_o[...] = sc[:, 128:]


def _tc_l1(n, d_in, h):
    grid = n // _R
    return pl.pallas_call(
        _l1_body,
        grid=(grid,),
        in_specs=[
            pl.BlockSpec((_R, d_in), lambda i: (i, 0)),
            pl.BlockSpec((_R, d_in), lambda i: (i, 0)),
            pl.BlockSpec((_R, d_in), lambda i: (i, 0)),
            pl.BlockSpec((_R, 1), lambda i: (i, 0)),
            pl.BlockSpec((d_in, h), lambda i: (0, 0)),
            pl.BlockSpec((1, h), lambda i: (0, 0)),
            pl.BlockSpec((h, h), lambda i: (0, 0)),
        ],
        out_specs=[
            pl.BlockSpec((_R, h), lambda i: (i, 0)),
            pl.BlockSpec((_R, 128), lambda i: (i, 0)),
            pl.BlockSpec((_R, 128), lambda i: (i, 0)),
        ],
        out_shape=[
            jax.ShapeDtypeStruct((n, h), jnp.float32),
            jax.ShapeDtypeStruct((n, 128), jnp.float32),
            jax.ShapeDtypeStruct((n, 128), jnp.float32),
        ],
    )


def _mid_body(sa, sb, hw, dis, b, wn, hwn_o, ha_o, hb_o):
    d = dis[...]
    s = jnp.concatenate([sa[...], sb[...]], axis=1)
    h = jnp.maximum(d * s + (d * d) * hw[...] + b[...], 0.0)
    hwn = lax.dot_general(h, wn[...], (((1,), (0,)), ((), ())),
                          preferred_element_type=jnp.float32)
    hwn_o[...] = hwn
    sc = d * hwn
    ha_o[...] = sc[:, :128]
    hb_o[...] = sc[:, 128:]


def _tc_mid(n, h):
    grid = n // _R
    return pl.pallas_call(
        _mid_body,
        grid=(grid,),
        in_specs=[
            pl.BlockSpec((_R, 128), lambda i: (i, 0)),
            pl.BlockSpec((_R, 128), lambda i: (i, 0)),
            pl.BlockSpec((_R, h), lambda i: (i, 0)),
            pl.BlockSpec((_R, 1), lambda i: (i, 0)),
            pl.BlockSpec((1, h), lambda i: (0, 0)),
            pl.BlockSpec((h, h), lambda i: (0, 0)),
        ],
        out_specs=[
            pl.BlockSpec((_R, h), lambda i: (i, 0)),
            pl.BlockSpec((_R, 128), lambda i: (i, 0)),
            pl.BlockSpec((_R, 128), lambda i: (i, 0)),
        ],
        out_shape=[
            jax.ShapeDtypeStruct((n, h), jnp.float32),
            jax.ShapeDtypeStruct((n, 128), jnp.float32),
            jax.ShapeDtypeStruct((n, 128), jnp.float32),
        ],
    )


def _final_body(sa, sb, hw, dis, b, batch, fw1, fb1, fw2, fb2, fw3, fb3,
                out_o, g_acc):
    d = dis[...]
    s = jnp.concatenate([sa[...], sb[...]], axis=1)
    h3 = jnp.maximum(d * s + (d * d) * hw[...] + b[...], 0.0)
    gidx = lax.broadcasted_iota(jnp.int32, (_R, 16), 1)
    onehot = (batch[...] == gidx).astype(jnp.float32)
    part = lax.dot_general(onehot, h3, (((0,), (0,)), ((), ())),
                           preferred_element_type=jnp.float32)
    i = pl.program_id(0)

    @pl.when(i == 0)
    def _():
        g_acc[...] = jnp.zeros_like(g_acc)

    g_acc[...] += part

    @pl.when(i == pl.num_programs(0) - 1)
    def _():
        g = g_acc[...]
        z = jnp.maximum(
            lax.dot_general(g, fw1[...], (((1,), (0,)), ((), ())),
                            preferred_element_type=jnp.float32) + fb1[...], 0.0)
        z = jnp.maximum(
            lax.dot_general(z, fw2[...], (((1,), (0,)), ((), ())),
                            preferred_element_type=jnp.float32) + fb2[...], 0.0)
        z = lax.dot_general(z, fw3[...], (((1,), (0,)), ((), ())),
                            preferred_element_type=jnp.float32) + fb3[...]
        m = jnp.max(z, axis=-1, keepdims=True)
        lse = jnp.log(jnp.sum(jnp.exp(z - m), axis=-1, keepdims=True)) + m
        out_o[...] = z - lse


def _tc_final(n, h, c, g):
    grid = n // _R
    return pl.pallas_call(
        _final_body,
        grid=(grid,),
        in_specs=[
            pl.BlockSpec((_R, 128), lambda i: (i, 0)),
            pl.BlockSpec((_R, 128), lambda i: (i, 0)),
            pl.BlockSpec((_R, h), lambda i: (i, 0)),
            pl.BlockSpec((_R, 1), lambda i: (i, 0)),
            pl.BlockSpec((1, h), lambda i: (0, 0)),
            pl.BlockSpec((_R, 1), lambda i: (i, 0)),
            pl.BlockSpec((h, h), lambda i: (0, 0)),
            pl.BlockSpec((1, h), lambda i: (0, 0)),
            pl.BlockSpec((h, h // 2), lambda i: (0, 0)),
            pl.BlockSpec((1, h // 2), lambda i: (0, 0)),
            pl.BlockSpec((h // 2, c), lambda i: (0, 0)),
            pl.BlockSpec((1, c), lambda i: (0, 0)),
        ],
        out_specs=pl.BlockSpec((g, c), lambda i: (0, 0)),
        out_shape=jax.ShapeDtypeStruct((g, c), jnp.float32),
        scratch_shapes=[pltpu.VMEM((g, h), jnp.float32)],
    )


def kernel(x, edge_index, batch, W1, b1, W2, b2, W3, b3,
           fW1, fb1, fW2, fb2, fW3, fb3):
    n, d_in = x.shape
    e = edge_index.shape[1]
    h = W1.shape[1]
    c = fW3.shape[1]
    g = 16
    stri = -(-(n // NS) // 8) * 8

    src2 = edge_index[0].reshape(e // CHUNK, CHUNK)
    dst2 = edge_index[1].reshape(e // CHUNK, CHUNK)
    ones128 = jnp.ones((CHUNK, 128), jnp.float32)
    zeros128 = jnp.zeros((stri, 128), jnp.float32)
    batch2 = batch.reshape(n, 1)

    d0, d1 = _make_deg(n, e)(dst2, zeros128, ones128)
    xs, dis = _tc_scale0(n, d_in)(d0, d1, x)

    s0a, s0b = _make_spmm(n, e, True)(xs, xs, src2, dst2, zeros128)
    hw2, h2a, h2b = _tc_l1(n, d_in, h)(s0a, s0b, x, dis, W1,
                                       b1.reshape(1, h), W2)

    s2a, s2b = _make_spmm(n, e, False)(h2a, h2b, src2, dst2, zeros128)
    hw3, h3a, h3b = _tc_mid(n, h)(s2a, s2b, hw2, dis, b2.reshape(1, h), W3)

    s3a, s3b = _make_spmm(n, e, False)(h3a, h3b, src2, dst2, zeros128)
    out = _tc_final(n, h, c, g)(s3a, s3b, hw3, dis, b3.reshape(1, h), batch2,
                                fW1, fb1.reshape(1, h), fW2,
                                fb2.reshape(1, h // 2), fW3,
                                fb3.reshape(1, c))
    return out


# speedup vs baseline: 20.1152x; 1.0000x over previous
import functools

import jax
import jax.numpy as jnp
from jax import lax
from jax.experimental import pallas as pl
from jax.experimental.pallas import tpu as pltpu
from jax.experimental.pallas import tpu_sc as plsc

NC = 2
NS = 16
NW = NC * NS
CHUNK = 80
SB = 32


def _split_chunks(total, workers, align=8):
    big = (-(-total // workers) + align - 1) // align * align
    for nbig in range(workers - 1, -1, -1):
        rem = total - big * nbig
        if rem < 0 or workers == nbig:
            continue
        sml, r = divmod(rem, workers - nbig)
        if r == 0 and sml % align == 0 and 0 < sml <= big:
            return big, sml, nbig
    raise ValueError((total, workers, align))


def _make_deg(n, e):
    total = e // CHUNK
    big, sml, nbig = _split_chunks(total, NW)
    stri = -(-(n // NS) // 8) * 8
    last = n - (NS - 1) * stri
    mesh = plsc.VectorSubcoreMesh(core_axis_name="c", subcore_axis_name="s",
                                  num_cores=NC, num_subcores=NS)
    out_sds = jax.ShapeDtypeStruct((n, 128), jnp.float32)

    @functools.partial(
        pl.kernel,
        out_type=(out_sds, out_sds),
        mesh=mesh,
        scratch_types=[
            pltpu.VMEM((big, CHUNK), jnp.int32),
            pltpu.VMEM((CHUNK, 128), jnp.float32),
            pltpu.VMEM_SHARED((n, 128), jnp.float32),
            pltpu.SemaphoreType.DMA,
            pltpu.SemaphoreType.DMA,
        ],
    )
    def deg_kernel(dst2_hbm, zeros_hbm, ones_hbm, d0_hbm, d1_hbm,
                   di_v, ones_v, acc_sh, ss0, ss1):
        c = lax.axis_index("c")
        s = lax.axis_index("s")
        w = s * NC + c
        base = s * stri
        ss = (ss0, ss1)

        @pl.when(s < NS - 1)
        def _():
            pltpu.sync_copy(zeros_hbm, acc_sh.at[pl.ds(base, stri)])

        @pl.when(s == NS - 1)
        def _():
            pltpu.sync_copy(zeros_hbm.at[pl.ds(0, last)],
                            acc_sh.at[pl.ds(base, last)])

        pltpu.sync_copy(ones_hbm, ones_v)

        @pl.when(w < nbig)
        def _():
            pltpu.sync_copy(dst2_hbm.at[pl.ds(w * big, big)], di_v)

        @pl.when(w >= nbig)
        def _():
            pltpu.sync_copy(
                dst2_hbm.at[pl.ds(nbig * big + (w - nbig) * sml, sml)],
                di_v.at[pl.ds(0, sml)])

        plsc.subcore_barrier()
        nch = jnp.where(w < nbig, big, sml)

        def body(t, carry):
            for b in (0, 1):
                j = 2 * t + b

                @pl.when(j >= 2)
                def _(b=b):
                    pltpu.make_async_copy(ones_hbm, ones_v, ss[b]).wait()

                pltpu.async_copy(ones_v, acc_sh.at[di_v.at[j]], ss[b],
                                 add=True)
            return carry

        lax.fori_loop(0, nch // 2, body, 0)
        pltpu.make_async_copy(ones_hbm, ones_v, ss0).wait()
        pltpu.make_async_copy(ones_hbm, ones_v, ss1).wait()
        plsc.subcore_barrier()
        for cc, o_hbm in ((0, d0_hbm), (1, d1_hbm)):
            for sz, cond in ((stri, (c == cc) & (s < NS - 1)),
                             (last, (c == cc) & (s == NS - 1))):
                @pl.when(cond)
                def _(sz=sz, o_hbm=o_hbm):
                    pltpu.sync_copy(acc_sh.at[pl.ds(base, sz)],
                                    o_hbm.at[pl.ds(base, sz)])

    return deg_kernel


def _make_spmm(n, e, split_edges_by_core):
    total = e // CHUNK
    big, sml, nbig = _split_chunks(total, NW if split_edges_by_core else NS,
                                   align=SB)
    stri = -(-(n // NS) // 8) * 8
    last = n - (NS - 1) * stri
    mesh = plsc.VectorSubcoreMesh(core_axis_name="c", subcore_axis_name="s",
                                  num_cores=NC, num_subcores=NS)
    out_sds = jax.ShapeDtypeStruct((n, 128), jnp.float32)

    @functools.partial(
        pl.kernel,
        out_type=(out_sds, out_sds),
        mesh=mesh,
        scratch_types=[
            pltpu.VMEM((SB, CHUNK), jnp.int32),
            pltpu.VMEM((SB, CHUNK), jnp.int32),
            pltpu.VMEM((CHUNK, 128), jnp.float32),
            pltpu.VMEM((CHUNK, 128), jnp.float32),
            pltpu.VMEM_SHARED((n, 128), jnp.float32),
            pltpu.SemaphoreType.DMA,
            pltpu.SemaphoreType.DMA,
            pltpu.SemaphoreType.DMA,
            pltpu.SemaphoreType.DMA,
        ],
    )
    def spmm_kernel(ha_hbm, hb_hbm, src2_hbm, dst2_hbm, zeros_hbm,
                    oa_hbm, ob_hbm, si_v, di_v, r0, r1, acc_sh,
                    sg0, sg1, ss0, ss1):
        c = lax.axis_index("c")
        s = lax.axis_index("s")
        w = s * NC + c if split_edges_by_core else s
        base = s * stri
        rows = (r0, r1)
        sg = (sg0, sg1)
        ss = (ss0, ss1)

        @pl.when(s < NS - 1)
        def _():
            pltpu.sync_copy(zeros_hbm, acc_sh.at[pl.ds(base, stri)])

        @pl.when(s == NS - 1)
        def _():
            pltpu.sync_copy(zeros_hbm.at[pl.ds(0, last)],
                            acc_sh.at[pl.ds(base, last)])

        plsc.subcore_barrier()
        nch = jnp.where(w < nbig, big, sml)
        rowbase = jnp.where(w < nbig, w * big,
                            nbig * big + (w - nbig) * sml)

        def edge_loop(h_hbm):
            def gather(j, b):
                pltpu.async_copy(h_hbm.at[si_v.at[j]], rows[b], sg[b])

            def gather_wait(b):
                pltpu.make_async_copy(h_hbm.at[pl.ds(0, CHUNK)], rows[b],
                                      sg[b]).wait()

            def scatter(j, b):
                pltpu.async_copy(rows[b], acc_sh.at[di_v.at[j]], ss[b],
                                 add=True)

            def scatter_wait(b):
                pltpu.make_async_copy(h_hbm.at[pl.ds(0, CHUNK)], rows[b],
                                      ss[b]).wait()

            def super_body(u, carry):
                rb = rowbase + u * SB
                pltpu.sync_copy(src2_hbm.at[pl.ds(rb, SB)], si_v)
                pltpu.sync_copy(dst2_hbm.at[pl.ds(rb, SB)], di_v)
                gather(0, 0)

                def body(t, carry2):
                    for b in (0, 1):
                        j = 2 * t + b

                        @pl.when(j >= 1)
                        def _(b=b):
                            scatter_wait(b ^ 1)

                        @pl.when(j + 1 < SB)
                        def _(j=j, b=b):
                            gather(j + 1, b ^ 1)

                        gather_wait(b)
                        scatter(j, b)
                    return carry2

                lax.fori_loop(0, SB // 2, body, 0)
                scatter_wait(1)
                return carry

            lax.fori_loop(0, nch // SB, super_body, 0)

        @pl.when(c == 0)
        def _():
            edge_loop(ha_hbm)

        @pl.when(c == 1)
        def _():
            edge_loop(hb_hbm)

        plsc.subcore_barrier()
        for cc, o_hbm in ((0, oa_hbm), (1, ob_hbm)):
            for sz, cond in ((stri, (c == cc) & (s < NS - 1)),
                             (last, (c == cc) & (s == NS - 1))):
                @pl.when(cond)
                def _(sz=sz, o_hbm=o_hbm):
                    pltpu.sync_copy(acc_sh.at[pl.ds(base, sz)],
                                    o_hbm.at[pl.ds(base, sz)])

    return spmm_kernel


_R = 2000


def _mm(a, b):
    return lax.dot_general(a.astype(jnp.bfloat16), b.astype(jnp.bfloat16),
                           (((1,), (0,)), ((), ())),
                           preferred_element_type=jnp.float32)


def _scale0_body(d0, d1, x, xs_o, dis_o):
    deg = d0[:, :1] + d1[:, :1] + 1.0
    dis = lax.rsqrt(deg)
    dis_o[...] = dis
    xs_o[...] = x[...] * dis


def _tc_scale0(n, d_in):
    grid = n // _R
    return pl.pallas_call(
        _scale0_body,
        grid=(grid,),
        in_specs=[
            pl.BlockSpec((_R, 128), lambda i: (i, 0)),
            pl.BlockSpec((_R, 128), lambda i: (i, 0)),
            pl.BlockSpec((_R, d_in), lambda i: (i, 0)),
        ],
        out_specs=[
            pl.BlockSpec((_R, d_in), lambda i: (i, 0)),
            pl.BlockSpec((_R, 1), lambda i: (i, 0)),
        ],
        out_shape=[
            jax.ShapeDtypeStruct((n, d_in), jnp.float32),
            jax.ShapeDtypeStruct((n, 1), jnp.float32),
        ],
    )


def _l1_body(sa, sb, x, dis, w1, b1, w2, hw2_o, ha_o, hb_o):
    d = dis[...]
    q = d * (sa[...] + sb[...]) + (d * d) * x[...]
    h1 = jnp.maximum(
        _mm(q, w1[...]) + b1[...], 0.0)
    hw2 = _mm(h1, w2[...])
    hw2_o[...] = hw2
    sc = d * hw2
    ha_o[...] = sc[:, :128]
    hb_o[...] = sc[:, 128:]


def _tc_l1(n, d_in, h):
    grid = n // _R
    return pl.pallas_call(
        _l1_body,
        grid=(grid,),
        in_specs=[
            pl.BlockSpec((_R, d_in), lambda i: (i, 0)),
            pl.BlockSpec((_R, d_in), lambda i: (i, 0)),
            pl.BlockSpec((_R, d_in), lambda i: (i, 0)),
            pl.BlockSpec((_R, 1), lambda i: (i, 0)),
            pl.BlockSpec((d_in, h), lambda i: (0, 0)),
            pl.BlockSpec((1, h), lambda i: (0, 0)),
            pl.BlockSpec((h, h), lambda i: (0, 0)),
        ],
        out_specs=[
            pl.BlockSpec((_R, h), lambda i: (i, 0)),
            pl.BlockSpec((_R, 128), lambda i: (i, 0)),
            pl.BlockSpec((_R, 128), lambda i: (i, 0)),
        ],
        out_shape=[
            jax.ShapeDtypeStruct((n, h), jnp.float32),
            jax.ShapeDtypeStruct((n, 128), jnp.float32),
            jax.ShapeDtypeStruct((n, 128), jnp.float32),
        ],
    )


def _mid_body(sa, sb, hw, dis, b, wn, hwn_o, ha_o, hb_o):
    d = dis[...]
    s = jnp.concatenate([sa[...], sb[...]], axis=1)
    h = jnp.maximum(d * s + (d * d) * hw[...] + b[...], 0.0)
    hwn = _mm(h, wn[...])
    hwn_o[...] = hwn
    sc = d * hwn
    ha_o[...] = sc[:, :128]
    hb_o[...] = sc[:, 128:]


def _tc_mid(n, h):
    grid = n // _R
    return pl.pallas_call(
        _mid_body,
        grid=(grid,),
        in_specs=[
            pl.BlockSpec((_R, 128), lambda i: (i, 0)),
            pl.BlockSpec((_R, 128), lambda i: (i, 0)),
            pl.BlockSpec((_R, h), lambda i: (i, 0)),
            pl.BlockSpec((_R, 1), lambda i: (i, 0)),
            pl.BlockSpec((1, h), lambda i: (0, 0)),
            pl.BlockSpec((h, h), lambda i: (0, 0)),
        ],
        out_specs=[
            pl.BlockSpec((_R, h), lambda i: (i, 0)),
            pl.BlockSpec((_R, 128), lambda i: (i, 0)),
            pl.BlockSpec((_R, 128), lambda i: (i, 0)),
        ],
        out_shape=[
            jax.ShapeDtypeStruct((n, h), jnp.float32),
            jax.ShapeDtypeStruct((n, 128), jnp.float32),
            jax.ShapeDtypeStruct((n, 128), jnp.float32),
        ],
    )


def _final_body(sa, sb, hw, dis, b, batch, fw1, fb1, fw2, fb2, fw3, fb3,
                out_o, g_acc):
    d = dis[...]
    s = jnp.concatenate([sa[...], sb[...]], axis=1)
    h3 = jnp.maximum(d * s + (d * d) * hw[...] + b[...], 0.0)
    gidx = lax.broadcasted_iota(jnp.int32, (_R, 16), 1)
    onehot = (batch[...] == gidx).astype(jnp.float32)
    part = lax.dot_general(onehot.astype(jnp.bfloat16),
                           h3.astype(jnp.bfloat16), (((0,), (0,)), ((), ())),
                           preferred_element_type=jnp.float32)
    i = pl.program_id(0)

    @pl.when(i == 0)
    def _():
        g_acc[...] = jnp.zeros_like(g_acc)

    g_acc[...] += part

    @pl.when(i == pl.num_programs(0) - 1)
    def _():
        g = g_acc[...]
        z = jnp.maximum(
            _mm(g, fw1[...]) + fb1[...], 0.0)
        z = jnp.maximum(
            _mm(z, fw2[...]) + fb2[...], 0.0)
        z = _mm(z, fw3[...]) + fb3[...]
        m = jnp.max(z, axis=-1, keepdims=True)
        lse = jnp.log(jnp.sum(jnp.exp(z - m), axis=-1, keepdims=True)) + m
        out_o[...] = z - lse


def _tc_final(n, h, c, g):
    grid = n // _R
    return pl.pallas_call(
        _final_body,
        grid=(grid,),
        in_specs=[
            pl.BlockSpec((_R, 128), lambda i: (i, 0)),
            pl.BlockSpec((_R, 128), lambda i: (i, 0)),
            pl.BlockSpec((_R, h), lambda i: (i, 0)),
            pl.BlockSpec((_R, 1), lambda i: (i, 0)),
            pl.BlockSpec((1, h), lambda i: (0, 0)),
            pl.BlockSpec((_R, 1), lambda i: (i, 0)),
            pl.BlockSpec((h, h), lambda i: (0, 0)),
            pl.BlockSpec((1, h), lambda i: (0, 0)),
            pl.BlockSpec((h, h // 2), lambda i: (0, 0)),
            pl.BlockSpec((1, h // 2), lambda i: (0, 0)),
            pl.BlockSpec((h // 2, c), lambda i: (0, 0)),
            pl.BlockSpec((1, c), lambda i: (0, 0)),
        ],
        out_specs=pl.BlockSpec((g, c), lambda i: (0, 0)),
        out_shape=jax.ShapeDtypeStruct((g, c), jnp.float32),
        scratch_shapes=[pltpu.VMEM((g, h), jnp.float32)],
    )


def kernel(x, edge_index, batch, W1, b1, W2, b2, W3, b3,
           fW1, fb1, fW2, fb2, fW3, fb3):
    n, d_in = x.shape
    e = edge_index.shape[1]
    h = W1.shape[1]
    c = fW3.shape[1]
    g = 16
    stri = -(-(n // NS) // 8) * 8

    src2 = edge_index[0].reshape(e // CHUNK, CHUNK)
    dst2 = edge_index[1].reshape(e // CHUNK, CHUNK)
    ones128 = jnp.ones((CHUNK, 128), jnp.float32)
    zeros128 = jnp.zeros((stri, 128), jnp.float32)
    batch2 = batch.reshape(n, 1)

    d0, d1 = _make_deg(n, e)(dst2, zeros128, ones128)
    xs, dis = _tc_scale0(n, d_in)(d0, d1, x)

    s0a, s0b = _make_spmm(n, e, True)(xs, xs, src2, dst2, zeros128)
    hw2, h2a, h2b = _tc_l1(n, d_in, h)(s0a, s0b, x, dis, W1,
                                       b1.reshape(1, h), W2)

    s2a, s2b = _make_spmm(n, e, False)(h2a, h2b, src2, dst2, zeros128)
    hw3, h3a, h3b = _tc_mid(n, h)(s2a, s2b, hw2, dis, b2.reshape(1, h), W3)

    s3a, s3b = _make_spmm(n, e, False)(h3a, h3b, src2, dst2, zeros128)
    out = _tc_final(n, h, c, g)(s3a, s3b, hw3, dis, b3.reshape(1, h), batch2,
                                fW1, fb1.reshape(1, h), fW2,
                                fb2.reshape(1, h // 2), fW3,
                                fb3.reshape(1, c))
    return out


# speedup vs baseline: 21.7067x; 1.0791x over previous
import functools

import jax
import jax.numpy as jnp
from jax import lax
from jax.experimental import pallas as pl
from jax.experimental.pallas import tpu as pltpu
from jax.experimental.pallas import tpu_sc as plsc

NC = 2
NS = 16
NW = NC * NS
CHUNK = 80
SB = 32


def _split_chunks(total, workers, align=8):
    big = (-(-total // workers) + align - 1) // align * align
    for nbig in range(workers - 1, -1, -1):
        rem = total - big * nbig
        if rem < 0 or workers == nbig:
            continue
        sml, r = divmod(rem, workers - nbig)
        if r == 0 and sml % align == 0 and 0 < sml <= big:
            return big, sml, nbig
    raise ValueError((total, workers, align))


def _make_deg(n, e):
    total = e // CHUNK
    big, sml, nbig = _split_chunks(total, NW)
    stri = -(-(n // NS) // 8) * 8
    last = n - (NS - 1) * stri
    mesh = plsc.VectorSubcoreMesh(core_axis_name="c", subcore_axis_name="s",
                                  num_cores=NC, num_subcores=NS)
    out_sds = jax.ShapeDtypeStruct((n, 128), jnp.float32)

    @functools.partial(
        pl.kernel,
        out_type=(out_sds, out_sds),
        mesh=mesh,
        scratch_types=[
            pltpu.VMEM((big, CHUNK), jnp.int32),
            pltpu.VMEM((CHUNK, 128), jnp.float32),
            pltpu.VMEM_SHARED((n, 128), jnp.float32),
            pltpu.SemaphoreType.DMA,
            pltpu.SemaphoreType.DMA,
        ],
    )
    def deg_kernel(dst2_hbm, zeros_hbm, ones_hbm, d0_hbm, d1_hbm,
                   di_v, ones_v, acc_sh, ss0, ss1):
        c = lax.axis_index("c")
        s = lax.axis_index("s")
        w = s * NC + c
        base = s * stri
        ss = (ss0, ss1)

        @pl.when(s < NS - 1)
        def _():
            pltpu.sync_copy(zeros_hbm, acc_sh.at[pl.ds(base, stri)])

        @pl.when(s == NS - 1)
        def _():
            pltpu.sync_copy(zeros_hbm.at[pl.ds(0, last)],
                            acc_sh.at[pl.ds(base, last)])

        pltpu.sync_copy(ones_hbm, ones_v)

        @pl.when(w < nbig)
        def _():
            pltpu.sync_copy(dst2_hbm.at[pl.ds(w * big, big)], di_v)

        @pl.when(w >= nbig)
        def _():
            pltpu.sync_copy(
                dst2_hbm.at[pl.ds(nbig * big + (w - nbig) * sml, sml)],
                di_v.at[pl.ds(0, sml)])

        plsc.subcore_barrier()
        nch = jnp.where(w < nbig, big, sml)

        def body(t, carry):
            for b in (0, 1):
                j = 2 * t + b

                @pl.when(j >= 2)
                def _(b=b):
                    pltpu.make_async_copy(ones_hbm, ones_v, ss[b]).wait()

                pltpu.async_copy(ones_v, acc_sh.at[di_v.at[j]], ss[b],
                                 add=True)
            return carry

        lax.fori_loop(0, nch // 2, body, 0)
        pltpu.make_async_copy(ones_hbm, ones_v, ss0).wait()
        pltpu.make_async_copy(ones_hbm, ones_v, ss1).wait()
        plsc.subcore_barrier()
        for cc, o_hbm in ((0, d0_hbm), (1, d1_hbm)):
            for sz, cond in ((stri, (c == cc) & (s < NS - 1)),
                             (last, (c == cc) & (s == NS - 1))):
                @pl.when(cond)
                def _(sz=sz, o_hbm=o_hbm):
                    pltpu.sync_copy(acc_sh.at[pl.ds(base, sz)],
                                    o_hbm.at[pl.ds(base, sz)])

    return deg_kernel


def _make_spmm(n, e, split_edges_by_core, chunk=CHUNK, sb=SB):
    total = e // chunk
    big, sml, nbig = _split_chunks(total, NW if split_edges_by_core else NS,
                                   align=sb)
    stri = -(-(n // NS) // 8) * 8
    last = n - (NS - 1) * stri
    mesh = plsc.VectorSubcoreMesh(core_axis_name="c", subcore_axis_name="s",
                                  num_cores=NC, num_subcores=NS)
    out_sds = jax.ShapeDtypeStruct((n, 128), jnp.float32)

    @functools.partial(
        pl.kernel,
        out_type=(out_sds, out_sds),
        mesh=mesh,
        scratch_types=[
            pltpu.VMEM((sb, chunk), jnp.int32),
            pltpu.VMEM((sb, chunk), jnp.int32),
            pltpu.VMEM((chunk, 128), jnp.float32),
            pltpu.VMEM((chunk, 128), jnp.float32),
            pltpu.VMEM_SHARED((n, 128), jnp.float32),
            pltpu.SemaphoreType.DMA,
            pltpu.SemaphoreType.DMA,
            pltpu.SemaphoreType.DMA,
            pltpu.SemaphoreType.DMA,
        ],
    )
    def spmm_kernel(ha_hbm, hb_hbm, src2_hbm, dst2_hbm, zeros_hbm, dummy_hbm,
                    oa_hbm, ob_hbm, si_v, di_v, r0, r1, acc_sh,
                    sg0, sg1, ss0, ss1):
        c = lax.axis_index("c")
        s = lax.axis_index("s")
        w = s * NC + c if split_edges_by_core else s
        base = s * stri
        rows = (r0, r1)
        sg = (sg0, sg1)
        ss = (ss0, ss1)

        @pl.when(s < NS - 1)
        def _():
            pltpu.sync_copy(zeros_hbm, acc_sh.at[pl.ds(base, stri)])

        @pl.when(s == NS - 1)
        def _():
            pltpu.sync_copy(zeros_hbm.at[pl.ds(0, last)],
                            acc_sh.at[pl.ds(base, last)])

        plsc.subcore_barrier()
        nch = jnp.where(w < nbig, big, sml)
        rowbase = jnp.where(w < nbig, w * big,
                            nbig * big + (w - nbig) * sml)

        def edge_loop(h_hbm):
            def gather(j, b):
                pltpu.async_copy(h_hbm.at[si_v.at[j]], rows[b], sg[b])

            def gather_wait(b):
                pltpu.make_async_copy(dummy_hbm, rows[b], sg[b]).wait()

            def scatter(j, b):
                pltpu.async_copy(rows[b], acc_sh.at[di_v.at[j]], ss[b],
                                 add=True)

            def scatter_wait(b):
                pltpu.make_async_copy(dummy_hbm, rows[b], ss[b]).wait()

            def super_body(u, carry):
                rb = rowbase + u * sb
                pltpu.async_copy(src2_hbm.at[pl.ds(rb, sb)], si_v, sg[0])
                pltpu.async_copy(dst2_hbm.at[pl.ds(rb, sb)], di_v, sg[0])
                pltpu.make_async_copy(src2_hbm.at[pl.ds(rb, sb)], si_v,
                                      sg[0]).wait()
                pltpu.make_async_copy(dst2_hbm.at[pl.ds(rb, sb)], di_v,
                                      sg[0]).wait()
                gather(0, 0)

                def body(t, carry2):
                    for b in (0, 1):
                        j = 2 * t + b

                        @pl.when(j >= 1)
                        def _(b=b):
                            scatter_wait(b ^ 1)

                        @pl.when(j + 1 < sb)
                        def _(j=j, b=b):
                            gather(j + 1, b ^ 1)

                        gather_wait(b)
                        scatter(j, b)
                    return carry2

                lax.fori_loop(0, sb // 2, body, 0)
                scatter_wait(1)
                return carry

            lax.fori_loop(0, nch // sb, super_body, 0)

        @pl.when(c == 0)
        def _():
            edge_loop(ha_hbm)

        @pl.when(c == 1)
        def _():
            edge_loop(hb_hbm)

        plsc.subcore_barrier()
        for cc, o_hbm in ((0, oa_hbm), (1, ob_hbm)):
            for sz, cond in ((stri, (c == cc) & (s < NS - 1)),
                             (last, (c == cc) & (s == NS - 1))):
                @pl.when(cond)
                def _(sz=sz, o_hbm=o_hbm):
                    pltpu.sync_copy(acc_sh.at[pl.ds(base, sz)],
                                    o_hbm.at[pl.ds(base, sz)])

    return spmm_kernel


_R = 2000


def _mm(a, b):
    return lax.dot_general(a.astype(jnp.bfloat16), b.astype(jnp.bfloat16),
                           (((1,), (0,)), ((), ())),
                           preferred_element_type=jnp.float32)


def _scale0_body(d0, d1, x, xs_o, dis_o):
    deg = d0[:, :1] + d1[:, :1] + 1.0
    dis = lax.rsqrt(deg)
    dis_o[...] = dis
    xs_o[...] = x[...] * dis


def _tc_scale0(n, d_in):
    grid = n // _R
    return pl.pallas_call(
        _scale0_body,
        grid=(grid,),
        in_specs=[
            pl.BlockSpec((_R, 128), lambda i: (i, 0)),
            pl.BlockSpec((_R, 128), lambda i: (i, 0)),
            pl.BlockSpec((_R, d_in), lambda i: (i, 0)),
        ],
        out_specs=[
            pl.BlockSpec((_R, d_in), lambda i: (i, 0)),
            pl.BlockSpec((_R, 1), lambda i: (i, 0)),
        ],
        out_shape=[
            jax.ShapeDtypeStruct((n, d_in), jnp.float32),
            jax.ShapeDtypeStruct((n, 1), jnp.float32),
        ],
    )


def _l1_body(sa, sb, x, dis, w1, b1, w2, hw2_o, ha_o, hb_o):
    d = dis[...]
    q = d * (sa[...] + sb[...]) + (d * d) * x[...]
    h1 = jnp.maximum(
        _mm(q, w1[...]) + b1[...], 0.0)
    hw2 = _mm(h1, w2[...])
    hw2_o[...] = hw2
    sc = d * hw2
    ha_o[...] = sc[:, :128]
    hb_o[...] = sc[:, 128:]


def _tc_l1(n, d_in, h):
    grid = n // _R
    return pl.pallas_call(
        _l1_body,
        grid=(grid,),
        in_specs=[
            pl.BlockSpec((_R, d_in), lambda i: (i, 0)),
            pl.BlockSpec((_R, d_in), lambda i: (i, 0)),
            pl.BlockSpec((_R, d_in), lambda i: (i, 0)),
            pl.BlockSpec((_R, 1), lambda i: (i, 0)),
            pl.BlockSpec((d_in, h), lambda i: (0, 0)),
            pl.BlockSpec((1, h), lambda i: (0, 0)),
            pl.BlockSpec((h, h), lambda i: (0, 0)),
        ],
        out_specs=[
            pl.BlockSpec((_R, h), lambda i: (i, 0)),
            pl.BlockSpec((_R, 128), lambda i: (i, 0)),
            pl.BlockSpec((_R, 128), lambda i: (i, 0)),
        ],
        out_shape=[
            jax.ShapeDtypeStruct((n, h), jnp.float32),
            jax.ShapeDtypeStruct((n, 128), jnp.float32),
            jax.ShapeDtypeStruct((n, 128), jnp.float32),
        ],
    )


def _mid_body(sa, sb, hw, dis, b, wn, hwn_o, ha_o, hb_o):
    d = dis[...]
    s = jnp.concatenate([sa[...], sb[...]], axis=1)
    h = jnp.maximum(d * s + (d * d) * hw[...] + b[...], 0.0)
    hwn = _mm(h, wn[...])
    hwn_o[...] = hwn
    sc = d * hwn
    ha_o[...] = sc[:, :128]
    hb_o[...] = sc[:, 128:]


def _tc_mid(n, h):
    grid = n // _R
    return pl.pallas_call(
        _mid_body,
        grid=(grid,),
        in_specs=[
            pl.BlockSpec((_R, 128), lambda i: (i, 0)),
            pl.BlockSpec((_R, 128), lambda i: (i, 0)),
            pl.BlockSpec((_R, h), lambda i: (i, 0)),
            pl.BlockSpec((_R, 1), lambda i: (i, 0)),
            pl.BlockSpec((1, h), lambda i: (0, 0)),
            pl.BlockSpec((h, h), lambda i: (0, 0)),
        ],
        out_specs=[
            pl.BlockSpec((_R, h), lambda i: (i, 0)),
            pl.BlockSpec((_R, 128), lambda i: (i, 0)),
            pl.BlockSpec((_R, 128), lambda i: (i, 0)),
        ],
        out_shape=[
            jax.ShapeDtypeStruct((n, h), jnp.float32),
            jax.ShapeDtypeStruct((n, 128), jnp.float32),
            jax.ShapeDtypeStruct((n, 128), jnp.float32),
        ],
    )


def _final_body(sa, sb, hw, dis, b, batch, fw1, fb1, fw2, fb2, fw3, fb3,
                out_o, g_acc):
    d = dis[...]
    s = jnp.concatenate([sa[...], sb[...]], axis=1)
    h3 = jnp.maximum(d * s + (d * d) * hw[...] + b[...], 0.0)
    gidx = lax.broadcasted_iota(jnp.int32, (_R, 16), 1)
    onehot = (batch[...] == gidx).astype(jnp.float32)
    part = lax.dot_general(onehot.astype(jnp.bfloat16),
                           h3.astype(jnp.bfloat16), (((0,), (0,)), ((), ())),
                           preferred_element_type=jnp.float32)
    i = pl.program_id(0)

    @pl.when(i == 0)
    def _():
        g_acc[...] = jnp.zeros_like(g_acc)

    g_acc[...] += part

    @pl.when(i == pl.num_programs(0) - 1)
    def _():
        g = g_acc[...]
        z = jnp.maximum(
            _mm(g, fw1[...]) + fb1[...], 0.0)
        z = jnp.maximum(
            _mm(z, fw2[...]) + fb2[...], 0.0)
        z = _mm(z, fw3[...]) + fb3[...]
        m = jnp.max(z, axis=-1, keepdims=True)
        lse = jnp.log(jnp.sum(jnp.exp(z - m), axis=-1, keepdims=True)) + m
        out_o[...] = z - lse


def _tc_final(n, h, c, g):
    grid = n // _R
    return pl.pallas_call(
        _final_body,
        grid=(grid,),
        in_specs=[
            pl.BlockSpec((_R, 128), lambda i: (i, 0)),
            pl.BlockSpec((_R, 128), lambda i: (i, 0)),
            pl.BlockSpec((_R, h), lambda i: (i, 0)),
            pl.BlockSpec((_R, 1), lambda i: (i, 0)),
            pl.BlockSpec((1, h), lambda i: (0, 0)),
            pl.BlockSpec((_R, 1), lambda i: (i, 0)),
            pl.BlockSpec((h, h), lambda i: (0, 0)),
            pl.BlockSpec((1, h), lambda i: (0, 0)),
            pl.BlockSpec((h, h // 2), lambda i: (0, 0)),
            pl.BlockSpec((1, h // 2), lambda i: (0, 0)),
            pl.BlockSpec((h // 2, c), lambda i: (0, 0)),
            pl.BlockSpec((1, c), lambda i: (0, 0)),
        ],
        out_specs=pl.BlockSpec((g, c), lambda i: (0, 0)),
        out_shape=jax.ShapeDtypeStruct((g, c), jnp.float32),
        scratch_shapes=[pltpu.VMEM((g, h), jnp.float32)],
    )


def kernel(x, edge_index, batch, W1, b1, W2, b2, W3, b3,
           fW1, fb1, fW2, fb2, fW3, fb3):
    n, d_in = x.shape
    e = edge_index.shape[1]
    h = W1.shape[1]
    c = fW3.shape[1]
    g = 16
    stri = -(-(n // NS) // 8) * 8

    src2 = edge_index[0].reshape(e // CHUNK, CHUNK)
    dst2 = edge_index[1].reshape(e // CHUNK, CHUNK)
    srcm = edge_index[0].reshape(e // 100, 100)
    dstm = edge_index[1].reshape(e // 100, 100)
    ones128 = jnp.ones((CHUNK, 128), jnp.float32)
    zeros128 = jnp.zeros((stri, 128), jnp.float32)
    batch2 = batch.reshape(n, 1)

    d0, d1 = _make_deg(n, e)(dst2, zeros128, ones128)
    xs, dis = _tc_scale0(n, d_in)(d0, d1, x)

    dummy80 = jnp.zeros((CHUNK, 128), jnp.float32)
    dummy100 = jnp.zeros((100, 128), jnp.float32)
    s0a, s0b = _make_spmm(n, e, True)(xs, xs, src2, dst2, zeros128, dummy80)
    hw2, h2a, h2b = _tc_l1(n, d_in, h)(s0a, s0b, x, dis, W1,
                                       b1.reshape(1, h), W2)

    s2a, s2b = _make_spmm(n, e, False, chunk=100, sb=40)(
        h2a, h2b, srcm, dstm, zeros128, dummy100)
    hw3, h3a, h3b = _tc_mid(n, h)(s2a, s2b, hw2, dis, b2.reshape(1, h), W3)

    s3a, s3b = _make_spmm(n, e, False, chunk=100, sb=40)(
        h3a, h3b, srcm, dstm, zeros128, dummy100)
    out = _tc_final(n, h, c, g)(s3a, s3b, hw3, dis, b3.reshape(1, h), batch2,
                                fW1, fb1.reshape(1, h), fW2,
                                fb2.reshape(1, h // 2), fW3,
                                fb3.reshape(1, c))
    return out
